# Initial kernel scaffold; baseline (speedup 1.0000x reference)
import jax
import jax.numpy as jnp
from jax.experimental import pallas as pl


def kernel(raw_src_seq, src_word_emb_weight):
    raise NotImplementedError("write your pallas kernel here")


# speedup vs baseline: 5.3685x; 5.3685x over previous
import functools

import jax
import jax.numpy as jnp
from jax import lax
from jax.experimental import pallas as pl
from jax.experimental.pallas import tpu as pltpu
from jax.experimental.pallas import tpu_sc as plsc

_N_VOCAB = 100000
_D = 128
_SCALE = float(_D) ** 0.5

_NC = 2
_NS = 16
_NW = _NC * _NS
_B = 4096 * 200
_BPW = _B // _NW
_CHUNK = 128
_NCHUNK = _BPW // _CHUNK


def _scale_body(t_ref, o_ref):
    o_ref[...] = t_ref[...] * _SCALE


def _scale_table(table):
    grid = 125
    blk = _N_VOCAB // grid
    return pl.pallas_call(
        _scale_body,
        out_shape=jax.ShapeDtypeStruct((_N_VOCAB, _D), jnp.float32),
        grid=(grid,),
        in_specs=[pl.BlockSpec((blk, _D), lambda i: (i, 0))],
        out_specs=pl.BlockSpec((blk, _D), lambda i: (i, 0)),
    )(table)


_mesh = plsc.VectorSubcoreMesh(core_axis_name="c", subcore_axis_name="s")


@functools.partial(
    pl.kernel,
    mesh=_mesh,
    out_type=jax.ShapeDtypeStruct((_NW, _NCHUNK, _CHUNK, _D), jnp.float32),
    scratch_types=[
        pltpu.VMEM((_NCHUNK, _CHUNK), jnp.int32),
        pltpu.VMEM((_CHUNK, _D), jnp.float32),
        pltpu.SemaphoreType.DMA,
    ],
)
def _sc_gather(table_hbm, idx_hbm, out_hbm, idx_v, buf, sem):
    wid = lax.axis_index("s") * _NC + lax.axis_index("c")
    pltpu.sync_copy(idx_hbm.at[wid], idx_v)

    def body(g, carry):
        pltpu.async_copy(table_hbm.at[idx_v.at[g]], buf, sem).wait()
        pltpu.sync_copy(buf, out_hbm.at[wid, g])
        return carry

    lax.fori_loop(0, _NCHUNK, body, 0)


def kernel(raw_src_seq, src_word_emb_weight):
    scaled = _scale_table(src_word_emb_weight)
    idx = raw_src_seq.astype(jnp.int32).reshape(_NW, _NCHUNK, _CHUNK)
    out = _sc_gather(scaled, idx)
    return out.reshape(4096, 200, _D)


# speedup vs baseline: 7.3023x; 1.3602x over previous
import functools

import jax
import jax.numpy as jnp
from jax import lax
from jax.experimental import pallas as pl
from jax.experimental.pallas import tpu as pltpu
from jax.experimental.pallas import tpu_sc as plsc

_N_VOCAB = 100000
_D = 128
_SCALE = float(_D) ** 0.5

_NC = 2
_NS = 16
_NW = _NC * _NS
_B = 4096 * 200
_BPW = _B // _NW
_CHUNK = 128
_NCHUNK = _BPW // _CHUNK


def _scale_body(t_ref, o_ref):
    o_ref[...] = t_ref[...] * _SCALE


def _scale_table(table):
    grid = 125
    blk = _N_VOCAB // grid
    return pl.pallas_call(
        _scale_body,
        out_shape=jax.ShapeDtypeStruct((_N_VOCAB, _D), jnp.float32),
        grid=(grid,),
        in_specs=[pl.BlockSpec((blk, _D), lambda i: (i, 0))],
        out_specs=pl.BlockSpec((blk, _D), lambda i: (i, 0)),
    )(table)


_mesh = plsc.VectorSubcoreMesh(core_axis_name="c", subcore_axis_name="s")

_NBUF = 4
_NITER = _NCHUNK // _NBUF


@functools.partial(
    pl.kernel,
    mesh=_mesh,
    out_type=jax.ShapeDtypeStruct((_NW, _NCHUNK, _CHUNK, _D), jnp.float32),
    scratch_types=[
        pltpu.VMEM((_NCHUNK, _CHUNK), jnp.int32),
        pltpu.VMEM((_NBUF, _CHUNK, _D), jnp.float32),
    ]
    + [pltpu.SemaphoreType.DMA] * (2 * _NBUF),
)
def _sc_gather(table_hbm, idx_hbm, out_hbm, idx_v, bufs, *sems):
    gsem = sems[:_NBUF]
    ssem = sems[_NBUF:]
    wid = lax.axis_index("s") * _NC + lax.axis_index("c")
    pltpu.sync_copy(idx_hbm.at[wid], idx_v)

    for b in range(_NBUF):
        pltpu.async_copy(table_hbm.at[idx_v.at[b]], bufs.at[b], gsem[b])

    def step(i, b, fire_next):
        g = i * _NBUF + b
        pltpu.make_async_copy(table_hbm.at[idx_v.at[g]], bufs.at[b], gsem[b]).wait()
        pltpu.async_copy(bufs.at[b], out_hbm.at[wid, g], ssem[b])
        pltpu.make_async_copy(bufs.at[b], out_hbm.at[wid, g], ssem[b]).wait()
        if fire_next:
            pltpu.async_copy(table_hbm.at[idx_v.at[g + _NBUF]], bufs.at[b], gsem[b])

    def body(i, carry):
        for b in range(_NBUF):
            step(i, b, True)
        return carry

    lax.fori_loop(0, _NITER - 1, body, 0)
    for b in range(_NBUF):
        step(_NITER - 1, b, False)


def kernel(raw_src_seq, src_word_emb_weight):
    scaled = _scale_table(src_word_emb_weight)
    idx = raw_src_seq.astype(jnp.int32).reshape(_NW, _NCHUNK, _CHUNK)
    out = _sc_gather(scaled, idx)
    return out.reshape(4096, 200, _D)


# speedup vs baseline: 8.2965x; 1.1361x over previous
import functools

import jax
import jax.numpy as jnp
from jax import lax
from jax.experimental import pallas as pl
from jax.experimental.pallas import tpu as pltpu
from jax.experimental.pallas import tpu_sc as plsc

_N_VOCAB = 100000
_D = 128
_SCALE = float(_D) ** 0.5

_NC = 2
_NS = 16
_NW = _NC * _NS
_B = 4096 * 200
_BPW = _B // _NW
_CHUNK = 128
_NCHUNK = _BPW // _CHUNK


def _scale_body(t_ref, o_ref):
    o_ref[...] = t_ref[...] * _SCALE


def _scale_table(table):
    grid = 25
    blk = _N_VOCAB // grid
    return pl.pallas_call(
        _scale_body,
        out_shape=jax.ShapeDtypeStruct((_N_VOCAB, _D), jnp.float32),
        grid=(grid,),
        in_specs=[pl.BlockSpec((blk, _D), lambda i: (i, 0))],
        out_specs=pl.BlockSpec((blk, _D), lambda i: (i, 0)),
    )(table)


_mesh = plsc.VectorSubcoreMesh(core_axis_name="c", subcore_axis_name="s")

_NBUF = 5
_NITER = _NCHUNK // _NBUF


@functools.partial(
    pl.kernel,
    mesh=_mesh,
    out_type=jax.ShapeDtypeStruct((_NW, _NCHUNK, _CHUNK, _D), jnp.float32),
    scratch_types=[
        pltpu.VMEM((_NCHUNK, _CHUNK), jnp.int32),
        pltpu.VMEM((_NBUF, _CHUNK, _D), jnp.float32),
    ]
    + [pltpu.SemaphoreType.DMA] * (2 * _NBUF),
)
def _sc_gather(table_hbm, idx_hbm, out_hbm, idx_v, bufs, *sems):
    gsem = sems[:_NBUF]
    ssem = sems[_NBUF:]
    wid = lax.axis_index("s") * _NC + lax.axis_index("c")
    pltpu.sync_copy(idx_hbm.at[wid], idx_v)

    for b in range(_NBUF):
        pltpu.async_copy(table_hbm.at[idx_v.at[b]], bufs.at[b], gsem[b])

    def step(i, b, fire_next):
        g = i * _NBUF + b
        pltpu.make_async_copy(table_hbm.at[idx_v.at[g]], bufs.at[b], gsem[b]).wait()
        pltpu.async_copy(bufs.at[b], out_hbm.at[wid, g], ssem[b])
        pltpu.make_async_copy(bufs.at[b], out_hbm.at[wid, g], ssem[b]).wait()
        if fire_next:
            pltpu.async_copy(table_hbm.at[idx_v.at[g + _NBUF]], bufs.at[b], gsem[b])

    def body(i, carry):
        for b in range(_NBUF):
            step(i, b, True)
        return carry

    lax.fori_loop(0, _NITER - 1, body, 0)
    for b in range(_NBUF):
        step(_NITER - 1, b, False)


def kernel(raw_src_seq, src_word_emb_weight):
    scaled = _scale_table(src_word_emb_weight)
    idx = raw_src_seq.astype(jnp.int32).reshape(_NW, _NCHUNK, _CHUNK)
    out = _sc_gather(scaled, idx)
    return out.reshape(4096, 200, _D)
